# baseline (speedup 1.0000x reference)
import functools

import jax
import jax.numpy as jnp
from jax import lax
from jax.experimental import pallas as pl
from jax.experimental.pallas import tpu as pltpu
from jax.experimental.pallas import tpu_sc as plsc

N = 10000
E = 320000
D = 128
N_PAD = 10240
NC = 2
NS = 16
NW = NC * NS
EPT = E // NW
BATCH = 125
NBATCH = EPT // BATCH
HHALF = N_PAD // 2
ROWS_PT = N_PAD // NS
RED_W = 2 * N_PAD // NS

_mesh = plsc.VectorSubcoreMesh(
    core_axis_name="c", subcore_axis_name="s", num_cores=NC, num_subcores=NS)


def _zeros16():
    return jnp.zeros((16,), jnp.float32)


_RCH = 320


def _deg_body(eidx, deg_out, idx_v, hist, seg_v, tmp_v, acc_v, shdeg):
    c = lax.axis_index("c")
    s = lax.axis_index("s")
    wid = s * NC + c
    lane = lax.iota(jnp.int32, 16)
    ones = jnp.ones((16,), jnp.float32)

    for arr in range(2):
        pltpu.sync_copy(eidx.at[arr, wid], idx_v)
        for base in (0, HHALF):
            def zh(j, _):
                for l in range(16):
                    hist[l, pl.ds(j * 16, 16)] = _zeros16()
                return _
            lax.fori_loop(0, HHALF // 16, zh, None)

            def sc(i, _):
                for u in range(5):
                    idx = idx_v[pl.ds((i * 5 + u) * 16, 16)]
                    m = (idx >= base) & (idx < base + HHALF)
                    li = jnp.where(m, idx - base, 0)
                    plsc.addupdate_scatter(hist, [lane, li], ones, mask=m)
                return _
            lax.fori_loop(0, EPT // 80, sc, None)

            def rd(j, _):
                acc = hist[0, pl.ds(j * 16, 16)]
                for l in range(1, 16):
                    acc = acc + hist[l, pl.ds(j * 16, 16)]
                seg_v[pl.ds(j * 16, 16)] = acc
                return _
            lax.fori_loop(0, HHALF // 16, rd, None)
            pltpu.sync_copy(seg_v, shdeg.at[s, pl.ds(arr * N_PAD + base,
                                                     HHALF)])

    plsc.subcore_barrier()

    col0 = s * RED_W
    for ch in range(RED_W // _RCH):
        pltpu.sync_copy(shdeg.at[:, pl.ds(col0 + ch * _RCH, _RCH)], tmp_v)

        def ad(j, _):
            acc = tmp_v[0, pl.ds(j * 16, 16)]
            for t in range(1, NS):
                acc = acc + tmp_v[t, pl.ds(j * 16, 16)]
            acc_v[pl.ds(ch * _RCH + j * 16, 16)] = acc
            return _
        lax.fori_loop(0, _RCH // 16, ad, None)

    pltpu.sync_copy(acc_v, deg_out.at[c, pl.ds(col0, RED_W)])


_deg_call = pl.kernel(
    _deg_body,
    out_type=jax.ShapeDtypeStruct((NC, 2 * N_PAD), jnp.float32),
    mesh=_mesh,
    compiler_params=pltpu.CompilerParams(use_tc_tiling_on_sc=False, needs_layout_passes=False),
    scratch_types=[
        pltpu.VMEM((EPT,), jnp.int32),
        pltpu.VMEM((NS, HHALF), jnp.float32),
        pltpu.VMEM((HHALF,), jnp.float32),
        pltpu.VMEM((NS, _RCH), jnp.float32),
        pltpu.VMEM((RED_W,), jnp.float32),
        pltpu.VMEM_SHARED((NS, 2 * N_PAD), jnp.float32),
    ],
)


def _feat_body(deg_ref, x_ref, w_ref, h2_ref, nd_ref):
    dg = deg_ref[0, :] + deg_ref[1, :]
    dsrc = dg[:N_PAD]
    ddst = dg[N_PAD:]
    ns = jnp.where(dsrc > 0, lax.rsqrt(jnp.maximum(dsrc, 1.0)), 0.0)
    nd = jnp.where(ddst > 0, lax.rsqrt(jnp.maximum(ddst, 1.0)), 0.0)
    h = x_ref[...] * ns[:, None]
    h2_ref[...] = jnp.dot(h, w_ref[...], preferred_element_type=jnp.float32)
    nd_ref[...] = nd[:, None]


_feat_call = pl.pallas_call(
    _feat_body,
    out_shape=(
        jax.ShapeDtypeStruct((N_PAD, D), jnp.float32),
        jax.ShapeDtypeStruct((N_PAD, 1), jnp.float32),
    ),
)


_DR = 32


def _msg_body(scd, h2, outp, P0, P1, rows0, rows1, dbuf,
              agg, sem0, sem1, semI0, semI1, semS0, semS1):
    c = lax.axis_index("c")
    s = lax.axis_index("s")
    wid = s * NC + c

    def zb(r, _):
        for k in range(D // 16):
            dbuf[r, pl.ds(k * 16, 16)] = _zeros16()
        return _
    lax.fori_loop(0, _DR, zb, None)
    row0 = s * ROWS_PT
    for k in range(ROWS_PT // _DR):
        pltpu.sync_copy(dbuf, agg.at[pl.ds(row0 + k * _DR, _DR)])
    plsc.subcore_barrier()

    pltpu.sync_copy(scd.at[wid, 0], P0)
    pltpu.sync_copy(scd.at[wid, 1], P1)
    pltpu.async_copy(h2.at[P0.at[0]], rows0, sem0)
    pltpu.async_copy(h2.at[P1.at[0]], rows1, sem1)

    def step(g, _):
        j0 = g * 2
        j1 = j0 + 1
        pltpu.make_async_copy(h2.at[P0.at[0]], rows0, sem0).wait()
        pltpu.make_async_copy(rows0, agg.at[P0.at[1]], semS0).start(add=True)
        pltpu.make_async_copy(h2.at[P1.at[0]], rows1, sem1).wait()
        pltpu.make_async_copy(rows1, agg.at[P1.at[1]], semS1).start(add=True)
        pltpu.make_async_copy(rows0, agg.at[P0.at[1]], semS0).wait()
        pltpu.async_copy(scd.at[wid, j0 + 2], P0, semI0)
        pltpu.make_async_copy(rows1, agg.at[P1.at[1]], semS1).wait()
        pltpu.async_copy(scd.at[wid, j1 + 2], P1, semI1)
        pltpu.make_async_copy(scd.at[wid, j0 + 2], P0, semI0).wait()
        pltpu.async_copy(h2.at[P0.at[0]], rows0, sem0)
        pltpu.make_async_copy(scd.at[wid, j1 + 2], P1, semI1).wait()
        pltpu.async_copy(h2.at[P1.at[0]], rows1, sem1)
        return _
    lax.fori_loop(0, NBATCH // 2, step, None)
    pltpu.make_async_copy(h2.at[P0.at[0]], rows0, sem0).wait()
    pltpu.make_async_copy(h2.at[P1.at[0]], rows1, sem1).wait()

    plsc.subcore_barrier()
    for k in range(ROWS_PT // _DR):
        pltpu.sync_copy(agg.at[pl.ds(row0 + k * _DR, _DR)], dbuf)
        pltpu.sync_copy(dbuf, outp.at[c, pl.ds(row0 + k * _DR, _DR)])


_msg_call = pl.kernel(
    _msg_body,
    out_type=jax.ShapeDtypeStruct((NC, N_PAD, D), jnp.float32),
    mesh=_mesh,
    compiler_params=pltpu.CompilerParams(use_tc_tiling_on_sc=False, needs_layout_passes=False),
    scratch_types=[
        pltpu.VMEM((2, BATCH), jnp.int32),
        pltpu.VMEM((2, BATCH), jnp.int32),
        pltpu.VMEM((BATCH, D), jnp.float32),
        pltpu.VMEM((BATCH, D), jnp.float32),
        pltpu.VMEM((_DR, D), jnp.float32),
        pltpu.VMEM_SHARED((N_PAD, D), jnp.float32),
        pltpu.SemaphoreType.DMA,
        pltpu.SemaphoreType.DMA,
        pltpu.SemaphoreType.DMA,
        pltpu.SemaphoreType.DMA,
        pltpu.SemaphoreType.DMA,
        pltpu.SemaphoreType.DMA,
    ],
)


def _out_body(agg_ref, nd_ref, b_ref, o_ref):
    a = agg_ref[0] + agg_ref[1]
    o_ref[...] = jnp.maximum(a * nd_ref[...] + b_ref[...], 0.0)


_OUT_ROWS = 1000

_out_call = pl.pallas_call(
    _out_body,
    grid=(N // _OUT_ROWS,),
    in_specs=[
        pl.BlockSpec((NC, _OUT_ROWS, D), lambda i: (0, i, 0)),
        pl.BlockSpec((_OUT_ROWS, 1), lambda i: (i, 0)),
        pl.BlockSpec((1, D), lambda i: (0, 0)),
    ],
    out_specs=pl.BlockSpec((_OUT_ROWS, D), lambda i: (i, 0)),
    out_shape=jax.ShapeDtypeStruct((N, D), jnp.float32),
)


@jax.jit
def kernel(x, edge_index, W, b):
    assert x.shape == (N, D) and edge_index.shape == (2, E)
    e32 = edge_index.astype(jnp.int32)
    eidx_a = e32.reshape(2, NW, EPT)
    scd = jnp.stack(
        [e32[0].reshape(NW, NBATCH, BATCH), e32[1].reshape(NW, NBATCH, BATCH)],
        axis=2)
    scd = jnp.concatenate([scd, scd[:, :2]], axis=1)
    x_pad = jnp.pad(x, ((0, N_PAD - N), (0, 0)))

    deg = _deg_call(eidx_a)
    h2, nd = _feat_call(deg, x_pad, W)
    aggp = _msg_call(scd, h2)
    out = _out_call(aggp, nd, b.reshape(1, D))
    return out


# speedup vs baseline: 1.1044x; 1.1044x over previous
import functools

import jax
import jax.numpy as jnp
from jax import lax
from jax.experimental import pallas as pl
from jax.experimental.pallas import tpu as pltpu
from jax.experimental.pallas import tpu_sc as plsc

N = 10000
E = 320000
D = 128
N_PAD = 10240
NC = 2
NS = 16
NW = NC * NS
EPT = E // NW
BATCH = 125
NBATCH = EPT // BATCH
HHALF = N_PAD // 2
ROWS_PT = N_PAD // NS
RED_W = 2 * N_PAD // NS

_mesh = plsc.VectorSubcoreMesh(
    core_axis_name="c", subcore_axis_name="s", num_cores=NC, num_subcores=NS)


def _zeros16():
    return jnp.zeros((16,), jnp.float32)


_RCH = 320


def _deg_body(eidx, deg_out, idx_v, hist, seg_v, tmp_v, acc_v, shdeg):
    c = lax.axis_index("c")
    s = lax.axis_index("s")
    wid = s * NC + c
    lane = lax.iota(jnp.int32, 16)
    ones = jnp.ones((16,), jnp.float32)

    for arr in range(2):
        pltpu.sync_copy(eidx.at[arr, wid], idx_v)
        for base in (0, HHALF):
            def zh(j, _):
                for l in range(16):
                    hist[l, pl.ds(j * 16, 16)] = _zeros16()
                return _
            lax.fori_loop(0, HHALF // 16, zh, None)

            def sc(i, _):
                for u in range(5):
                    idx = idx_v[pl.ds((i * 5 + u) * 16, 16)]
                    m = (idx >= base) & (idx < base + HHALF)
                    li = jnp.where(m, idx - base, 0)
                    plsc.addupdate_scatter(hist, [lane, li], ones, mask=m)
                return _
            lax.fori_loop(0, EPT // 80, sc, None)

            def rd(j, _):
                acc = hist[0, pl.ds(j * 16, 16)]
                for l in range(1, 16):
                    acc = acc + hist[l, pl.ds(j * 16, 16)]
                seg_v[pl.ds(j * 16, 16)] = acc
                return _
            lax.fori_loop(0, HHALF // 16, rd, None)
            pltpu.sync_copy(seg_v, shdeg.at[s, pl.ds(arr * N_PAD + base,
                                                     HHALF)])

    plsc.subcore_barrier()

    col0 = s * RED_W
    for ch in range(RED_W // _RCH):
        pltpu.sync_copy(shdeg.at[:, pl.ds(col0 + ch * _RCH, _RCH)], tmp_v)

        def ad(j, _):
            acc = tmp_v[0, pl.ds(j * 16, 16)]
            for t in range(1, NS):
                acc = acc + tmp_v[t, pl.ds(j * 16, 16)]
            acc_v[pl.ds(ch * _RCH + j * 16, 16)] = acc
            return _
        lax.fori_loop(0, _RCH // 16, ad, None)

    pltpu.sync_copy(acc_v, deg_out.at[c, pl.ds(col0, RED_W)])


_deg_call = pl.kernel(
    _deg_body,
    out_type=jax.ShapeDtypeStruct((NC, 2 * N_PAD), jnp.float32),
    mesh=_mesh,
    compiler_params=pltpu.CompilerParams(use_tc_tiling_on_sc=False, needs_layout_passes=False),
    scratch_types=[
        pltpu.VMEM((EPT,), jnp.int32),
        pltpu.VMEM((NS, HHALF), jnp.float32),
        pltpu.VMEM((HHALF,), jnp.float32),
        pltpu.VMEM((NS, _RCH), jnp.float32),
        pltpu.VMEM((RED_W,), jnp.float32),
        pltpu.VMEM_SHARED((NS, 2 * N_PAD), jnp.float32),
    ],
)


def _feat_body(deg_ref, x_ref, w_ref, h2_ref, nd_ref):
    dg = deg_ref[0, :] + deg_ref[1, :]
    dsrc = dg[:N_PAD]
    ddst = dg[N_PAD:]
    ns = jnp.where(dsrc > 0, lax.rsqrt(jnp.maximum(dsrc, 1.0)), 0.0)
    nd = jnp.where(ddst > 0, lax.rsqrt(jnp.maximum(ddst, 1.0)), 0.0)
    h = x_ref[...] * ns[:N, None]
    h2_ref[:N, :] = jnp.dot(h, w_ref[...], preferred_element_type=jnp.float32)
    h2_ref[N:, :] = jnp.zeros((N_PAD - N, D), jnp.float32)
    nd_ref[...] = nd[:, None]


_feat_call = pl.pallas_call(
    _feat_body,
    out_shape=(
        jax.ShapeDtypeStruct((N_PAD, D), jnp.float32),
        jax.ShapeDtypeStruct((N_PAD, 1), jnp.float32),
    ),
)


_DR = 32


def _msg_body(scd, h2, outp, P0, P1, rows0, rows1, dbuf,
              agg, sem0, sem1, semI0, semI1, semS0, semS1):
    c = lax.axis_index("c")
    s = lax.axis_index("s")
    wid = s * NC + c

    def zb(r, _):
        for k in range(D // 16):
            dbuf[r, pl.ds(k * 16, 16)] = _zeros16()
        return _
    lax.fori_loop(0, _DR, zb, None)
    row0 = s * ROWS_PT
    for k in range(ROWS_PT // _DR):
        pltpu.sync_copy(dbuf, agg.at[pl.ds(row0 + k * _DR, _DR)])
    plsc.subcore_barrier()

    pltpu.sync_copy(scd.at[wid, 0], P0)
    pltpu.sync_copy(scd.at[wid, 1], P1)
    pltpu.async_copy(h2.at[P0.at[0]], rows0, sem0)
    pltpu.async_copy(h2.at[P1.at[0]], rows1, sem1)

    def step(g, _):
        j0 = g * 2
        j1 = j0 + 1
        pltpu.make_async_copy(h2.at[P0.at[0]], rows0, sem0).wait()
        pltpu.sync_copy(rows0, agg.at[P0.at[1]], add=True)
        pltpu.async_copy(scd.at[wid, jnp.minimum(j0 + 2, NBATCH - 1)],
                         P0, semI0)
        pltpu.make_async_copy(h2.at[P1.at[0]], rows1, sem1).wait()
        pltpu.sync_copy(rows1, agg.at[P1.at[1]], add=True)
        pltpu.async_copy(scd.at[wid, jnp.minimum(j1 + 2, NBATCH - 1)],
                         P1, semI1)
        pltpu.make_async_copy(scd.at[wid, 0], P0, semI0).wait()
        pltpu.async_copy(h2.at[P0.at[0]], rows0, sem0)
        pltpu.make_async_copy(scd.at[wid, 0], P1, semI1).wait()
        pltpu.async_copy(h2.at[P1.at[0]], rows1, sem1)
        return _
    lax.fori_loop(0, NBATCH // 2, step, None)
    pltpu.make_async_copy(h2.at[P0.at[0]], rows0, sem0).wait()
    pltpu.make_async_copy(h2.at[P1.at[0]], rows1, sem1).wait()

    plsc.subcore_barrier()
    for k in range(ROWS_PT // _DR):
        pltpu.sync_copy(agg.at[pl.ds(row0 + k * _DR, _DR)], dbuf)
        pltpu.sync_copy(dbuf, outp.at[c, pl.ds(row0 + k * _DR, _DR)])


_msg_call = pl.kernel(
    _msg_body,
    out_type=jax.ShapeDtypeStruct((NC, N_PAD, D), jnp.float32),
    mesh=_mesh,
    compiler_params=pltpu.CompilerParams(use_tc_tiling_on_sc=False, needs_layout_passes=False),
    scratch_types=[
        pltpu.VMEM((2, BATCH), jnp.int32),
        pltpu.VMEM((2, BATCH), jnp.int32),
        pltpu.VMEM((BATCH, D), jnp.float32),
        pltpu.VMEM((BATCH, D), jnp.float32),
        pltpu.VMEM((_DR, D), jnp.float32),
        pltpu.VMEM_SHARED((N_PAD, D), jnp.float32),
        pltpu.SemaphoreType.DMA,
        pltpu.SemaphoreType.DMA,
        pltpu.SemaphoreType.DMA,
        pltpu.SemaphoreType.DMA,
        pltpu.SemaphoreType.DMA,
        pltpu.SemaphoreType.DMA,
    ],
)


def _out_body(agg_ref, nd_ref, b_ref, o_ref):
    a = agg_ref[0] + agg_ref[1]
    o_ref[...] = jnp.maximum(a * nd_ref[...] + b_ref[...], 0.0)


_OUT_ROWS = 1000

_out_call = pl.pallas_call(
    _out_body,
    grid=(N // _OUT_ROWS,),
    in_specs=[
        pl.BlockSpec((NC, _OUT_ROWS, D), lambda i: (0, i, 0)),
        pl.BlockSpec((_OUT_ROWS, 1), lambda i: (i, 0)),
        pl.BlockSpec((1, D), lambda i: (0, 0)),
    ],
    out_specs=pl.BlockSpec((_OUT_ROWS, D), lambda i: (i, 0)),
    out_shape=jax.ShapeDtypeStruct((N, D), jnp.float32),
)


@jax.jit
def kernel(x, edge_index, W, b):
    assert x.shape == (N, D) and edge_index.shape == (2, E)
    e32 = edge_index.astype(jnp.int32)
    eidx_a = e32.reshape(2, NW, EPT)
    scd = jnp.stack(
        [e32[0].reshape(NW, NBATCH, BATCH), e32[1].reshape(NW, NBATCH, BATCH)],
        axis=2)

    deg = _deg_call(eidx_a)
    h2, nd = _feat_call(deg, x, W)
    aggp = _msg_call(scd, h2)
    out = _out_call(aggp, nd, b.reshape(1, D))
    return out


# speedup vs baseline: 1.2319x; 1.1154x over previous
import functools

import jax
import jax.numpy as jnp
from jax import lax
from jax.experimental import pallas as pl
from jax.experimental.pallas import tpu as pltpu
from jax.experimental.pallas import tpu_sc as plsc

N = 10000
E = 320000
D = 128
N_PAD = 10240
NC = 2
NS = 16
NW = NC * NS
EPT = E // NW
BATCH = 125
NBATCH = EPT // BATCH
HHALF = N_PAD // 2
ROWS_PT = N_PAD // NS
RED_W = 2 * N_PAD // NS

_mesh = plsc.VectorSubcoreMesh(
    core_axis_name="c", subcore_axis_name="s", num_cores=NC, num_subcores=NS)


def _zeros16():
    return jnp.zeros((16,), jnp.float32)


_RCH = 320


def _deg_body(eidx, deg_out, idx_v, hist, seg_v, tmp_v, acc_v, shdeg):
    c = lax.axis_index("c")
    s = lax.axis_index("s")
    wid = s * NC + c
    lane = lax.iota(jnp.int32, 16)
    lane8 = lane & 7
    m_lo = lane < 8
    m_hi = lane >= 8
    ones = jnp.ones((16,), jnp.float32)

    for arr in range(2):
        pltpu.sync_copy(eidx.at[arr, wid], idx_v)
        def zh(j, _):
            for l in range(8):
                hist[l, pl.ds(j * 16, 16)] = _zeros16()
            return _
        lax.fori_loop(0, N_PAD // 16, zh, None)

        def sc(i, _):
            for u in range(5):
                idx = idx_v[pl.ds((i * 5 + u) * 16, 16)]
                plsc.addupdate_scatter(hist, [lane8, idx], ones, mask=m_lo)
                plsc.addupdate_scatter(hist, [lane8, idx], ones, mask=m_hi)
            return _
        lax.fori_loop(0, EPT // 80, sc, None)

        def rd(j, _):
            acc = hist[0, pl.ds(j * 16, 16)]
            for l in range(1, 8):
                acc = acc + hist[l, pl.ds(j * 16, 16)]
            seg_v[pl.ds(j * 16, 16)] = acc
            return _
        lax.fori_loop(0, N_PAD // 16, rd, None)
        pltpu.sync_copy(seg_v, shdeg.at[s, pl.ds(arr * N_PAD, N_PAD)])

    plsc.subcore_barrier()

    col0 = s * RED_W
    for ch in range(RED_W // _RCH):
        pltpu.sync_copy(shdeg.at[:, pl.ds(col0 + ch * _RCH, _RCH)], tmp_v)

        def ad(j, _):
            acc = tmp_v[0, pl.ds(j * 16, 16)]
            for t in range(1, NS):
                acc = acc + tmp_v[t, pl.ds(j * 16, 16)]
            acc_v[pl.ds(ch * _RCH + j * 16, 16)] = acc
            return _
        lax.fori_loop(0, _RCH // 16, ad, None)

    pltpu.sync_copy(acc_v, deg_out.at[c, pl.ds(col0, RED_W)])


_deg_call = pl.kernel(
    _deg_body,
    out_type=jax.ShapeDtypeStruct((NC, 2 * N_PAD), jnp.float32),
    mesh=_mesh,
    compiler_params=pltpu.CompilerParams(use_tc_tiling_on_sc=False, needs_layout_passes=False),
    scratch_types=[
        pltpu.VMEM((EPT,), jnp.int32),
        pltpu.VMEM((8, N_PAD), jnp.float32),
        pltpu.VMEM((N_PAD,), jnp.float32),
        pltpu.VMEM((NS, _RCH), jnp.float32),
        pltpu.VMEM((RED_W,), jnp.float32),
        pltpu.VMEM_SHARED((NS, 2 * N_PAD), jnp.float32),
    ],
)


def _feat_body(deg_ref, x_ref, w_ref, h2_ref, nd_ref):
    dg = deg_ref[0, :] + deg_ref[1, :]
    dsrc = dg[:N_PAD]
    ddst = dg[N_PAD:]
    ns = jnp.where(dsrc > 0, lax.rsqrt(jnp.maximum(dsrc, 1.0)), 0.0)
    nd = jnp.where(ddst > 0, lax.rsqrt(jnp.maximum(ddst, 1.0)), 0.0)
    h = x_ref[...] * ns[:N, None]
    h2_ref[:N, :] = jnp.dot(h, w_ref[...], preferred_element_type=jnp.float32)
    h2_ref[N:, :] = jnp.zeros((N_PAD - N, D), jnp.float32)
    nd_ref[...] = nd[:, None]


_feat_call = pl.pallas_call(
    _feat_body,
    out_shape=(
        jax.ShapeDtypeStruct((N_PAD, D), jnp.float32),
        jax.ShapeDtypeStruct((N_PAD, 1), jnp.float32),
    ),
)


_DR = 32


def _msg_body(scd, h2, outp, P0, P1, rows0, rows1, dbuf,
              agg, sem0, sem1, semI0, semI1, semS0, semS1):
    c = lax.axis_index("c")
    s = lax.axis_index("s")
    wid = s * NC + c

    def zb(r, _):
        for k in range(D // 16):
            dbuf[r, pl.ds(k * 16, 16)] = _zeros16()
        return _
    lax.fori_loop(0, _DR, zb, None)
    row0 = s * ROWS_PT
    for k in range(ROWS_PT // _DR):
        pltpu.sync_copy(dbuf, agg.at[pl.ds(row0 + k * _DR, _DR)])
    plsc.subcore_barrier()

    pltpu.sync_copy(scd.at[wid, 0], P0)
    pltpu.sync_copy(scd.at[wid, 1], P1)
    pltpu.async_copy(h2.at[P0.at[0]], rows0, sem0)
    pltpu.async_copy(h2.at[P1.at[0]], rows1, sem1)

    def step(g, _):
        j0 = g * 2
        j1 = j0 + 1
        pltpu.make_async_copy(h2.at[P0.at[0]], rows0, sem0).wait()
        pltpu.sync_copy(rows0, agg.at[P0.at[1]], add=True)
        pltpu.async_copy(scd.at[wid, jnp.minimum(j0 + 2, NBATCH - 1)],
                         P0, semI0)
        pltpu.make_async_copy(h2.at[P1.at[0]], rows1, sem1).wait()
        pltpu.sync_copy(rows1, agg.at[P1.at[1]], add=True)
        pltpu.async_copy(scd.at[wid, jnp.minimum(j1 + 2, NBATCH - 1)],
                         P1, semI1)
        pltpu.make_async_copy(scd.at[wid, 0], P0, semI0).wait()
        pltpu.async_copy(h2.at[P0.at[0]], rows0, sem0)
        pltpu.make_async_copy(scd.at[wid, 0], P1, semI1).wait()
        pltpu.async_copy(h2.at[P1.at[0]], rows1, sem1)
        return _
    lax.fori_loop(0, NBATCH // 2, step, None)
    pltpu.make_async_copy(h2.at[P0.at[0]], rows0, sem0).wait()
    pltpu.make_async_copy(h2.at[P1.at[0]], rows1, sem1).wait()

    plsc.subcore_barrier()
    for k in range(ROWS_PT // _DR):
        pltpu.sync_copy(agg.at[pl.ds(row0 + k * _DR, _DR)], dbuf)
        pltpu.sync_copy(dbuf, outp.at[c, pl.ds(row0 + k * _DR, _DR)])


_msg_call = pl.kernel(
    _msg_body,
    out_type=jax.ShapeDtypeStruct((NC, N_PAD, D), jnp.float32),
    mesh=_mesh,
    compiler_params=pltpu.CompilerParams(use_tc_tiling_on_sc=False, needs_layout_passes=False),
    scratch_types=[
        pltpu.VMEM((2, BATCH), jnp.int32),
        pltpu.VMEM((2, BATCH), jnp.int32),
        pltpu.VMEM((BATCH, D), jnp.float32),
        pltpu.VMEM((BATCH, D), jnp.float32),
        pltpu.VMEM((_DR, D), jnp.float32),
        pltpu.VMEM_SHARED((N_PAD, D), jnp.float32),
        pltpu.SemaphoreType.DMA,
        pltpu.SemaphoreType.DMA,
        pltpu.SemaphoreType.DMA,
        pltpu.SemaphoreType.DMA,
        pltpu.SemaphoreType.DMA,
        pltpu.SemaphoreType.DMA,
    ],
)


def _out_body(agg_ref, nd_ref, b_ref, o_ref):
    a = agg_ref[0] + agg_ref[1]
    o_ref[...] = jnp.maximum(a * nd_ref[...] + b_ref[...], 0.0)


_OUT_ROWS = 1000

_out_call = pl.pallas_call(
    _out_body,
    grid=(N // _OUT_ROWS,),
    in_specs=[
        pl.BlockSpec((NC, _OUT_ROWS, D), lambda i: (0, i, 0)),
        pl.BlockSpec((_OUT_ROWS, 1), lambda i: (i, 0)),
        pl.BlockSpec((1, D), lambda i: (0, 0)),
    ],
    out_specs=pl.BlockSpec((_OUT_ROWS, D), lambda i: (i, 0)),
    out_shape=jax.ShapeDtypeStruct((N, D), jnp.float32),
)


@jax.jit
def kernel(x, edge_index, W, b):
    assert x.shape == (N, D) and edge_index.shape == (2, E)
    e32 = edge_index.astype(jnp.int32)
    eidx_a = e32.reshape(2, NW, EPT)
    scd = jnp.stack(
        [e32[0].reshape(NW, NBATCH, BATCH), e32[1].reshape(NW, NBATCH, BATCH)],
        axis=2)

    deg = _deg_call(eidx_a)
    h2, nd = _feat_call(deg, x, W)
    aggp = _msg_call(scd, h2)
    out = _out_call(aggp, nd, b.reshape(1, D))
    return out


# speedup vs baseline: 1.2432x; 1.0092x over previous
import functools

import jax
import jax.numpy as jnp
from jax import lax
from jax.experimental import pallas as pl
from jax.experimental.pallas import tpu as pltpu
from jax.experimental.pallas import tpu_sc as plsc

N = 10000
E = 320000
D = 128
N_PAD = 10240
NC = 2
NS = 16
NW = NC * NS
EPT = E // NW
BATCH = 125
NBATCH = EPT // BATCH
HHALF = N_PAD // 2
ROWS_PT = N_PAD // NS
RED_W = 2 * N_PAD // NS

_mesh = plsc.VectorSubcoreMesh(
    core_axis_name="c", subcore_axis_name="s", num_cores=NC, num_subcores=NS)


def _zeros16():
    return jnp.zeros((16,), jnp.float32)


_RCH = 320


def _deg_body(eidx, deg_out, idx_v, hist, seg_v, tmp_v, acc_v, shdeg):
    c = lax.axis_index("c")
    s = lax.axis_index("s")
    wid = s * NC + c
    lane = lax.iota(jnp.int32, 16)
    lane8 = lane & 7
    m_lo = lane < 8
    m_hi = lane >= 8
    ones = jnp.ones((16,), jnp.float32)

    for arr in range(2):
        pltpu.sync_copy(eidx.at[arr, wid], idx_v)
        def zh(j, _):
            for l in range(8):
                hist[l, pl.ds(j * 16, 16)] = _zeros16()
            return _
        lax.fori_loop(0, N_PAD // 16, zh, None)

        def sc(i, _):
            for u in range(5):
                idx = idx_v[pl.ds((i * 5 + u) * 16, 16)]
                plsc.addupdate_scatter(hist, [lane8, idx], ones, mask=m_lo)
                plsc.addupdate_scatter(hist, [lane8, idx], ones, mask=m_hi)
            return _
        lax.fori_loop(0, EPT // 80, sc, None)

        def rd(j, _):
            acc = hist[0, pl.ds(j * 16, 16)]
            for l in range(1, 8):
                acc = acc + hist[l, pl.ds(j * 16, 16)]
            seg_v[pl.ds(j * 16, 16)] = acc
            return _
        lax.fori_loop(0, N_PAD // 16, rd, None)
        pltpu.sync_copy(seg_v, shdeg.at[s, pl.ds(arr * N_PAD, N_PAD)])

    plsc.subcore_barrier()

    col0 = s * RED_W
    for ch in range(RED_W // _RCH):
        pltpu.sync_copy(shdeg.at[:, pl.ds(col0 + ch * _RCH, _RCH)], tmp_v)

        def ad(j, _):
            acc = tmp_v[0, pl.ds(j * 16, 16)]
            for t in range(1, NS):
                acc = acc + tmp_v[t, pl.ds(j * 16, 16)]
            acc_v[pl.ds(ch * _RCH + j * 16, 16)] = acc
            return _
        lax.fori_loop(0, _RCH // 16, ad, None)

    pltpu.sync_copy(acc_v, deg_out.at[c, pl.ds(col0, RED_W)])


_deg_call = pl.kernel(
    _deg_body,
    out_type=jax.ShapeDtypeStruct((NC, 2 * N_PAD), jnp.float32),
    mesh=_mesh,
    compiler_params=pltpu.CompilerParams(use_tc_tiling_on_sc=False, needs_layout_passes=False),
    scratch_types=[
        pltpu.VMEM((EPT,), jnp.int32),
        pltpu.VMEM((8, N_PAD), jnp.float32),
        pltpu.VMEM((N_PAD,), jnp.float32),
        pltpu.VMEM((NS, _RCH), jnp.float32),
        pltpu.VMEM((RED_W,), jnp.float32),
        pltpu.VMEM_SHARED((NS, 2 * N_PAD), jnp.float32),
    ],
)


def _feat_body(deg_ref, x_ref, w_ref, h2_ref, nd_ref):
    dg = deg_ref[0, :] + deg_ref[1, :]
    dsrc = dg[:N_PAD]
    ddst = dg[N_PAD:]
    ns = jnp.where(dsrc > 0, lax.rsqrt(jnp.maximum(dsrc, 1.0)), 0.0)
    nd = jnp.where(ddst > 0, lax.rsqrt(jnp.maximum(ddst, 1.0)), 0.0)
    h = x_ref[...] * ns[:N, None]
    h2_ref[:N, :] = jnp.dot(h, w_ref[...], preferred_element_type=jnp.float32)
    h2_ref[N:, :] = jnp.zeros((N_PAD - N, D), jnp.float32)
    nd_ref[...] = nd[:, None]


_feat_call = pl.pallas_call(
    _feat_body,
    out_shape=(
        jax.ShapeDtypeStruct((N_PAD, D), jnp.float32),
        jax.ShapeDtypeStruct((N_PAD, 1), jnp.float32),
    ),
)


_DR = 32


def _msg_body(scd, h2, outp, P0, P1, rows0, rows1, dbuf,
              agg, sem0, sem1, semI0, semI1, semS0, semS1):
    c = lax.axis_index("c")
    s = lax.axis_index("s")
    wid = s * NC + c

    def zb(r, _):
        for k in range(D // 16):
            dbuf[r, pl.ds(k * 16, 16)] = _zeros16()
        return _
    lax.fori_loop(0, _DR, zb, None)
    row0 = s * ROWS_PT
    for k in range(ROWS_PT // _DR):
        pltpu.sync_copy(dbuf, agg.at[pl.ds(row0 + k * _DR, _DR)])
    plsc.subcore_barrier()

    pltpu.sync_copy(scd.at[wid, 0], P0)
    pltpu.sync_copy(scd.at[wid, 1], P1)
    pltpu.async_copy(h2.at[P0.at[0]], rows0, sem0)
    pltpu.async_copy(h2.at[P1.at[0]], rows1, sem1)

    def step(g, _):
        j0 = g * 2
        j1 = j0 + 1
        pltpu.make_async_copy(h2.at[P0.at[0]], rows0, sem0).wait()
        pltpu.sync_copy(rows0, agg.at[P0.at[1]], add=True)
        pltpu.async_copy(scd.at[wid, jnp.minimum(j0 + 2, NBATCH - 1)],
                         P0, semI0)
        pltpu.make_async_copy(h2.at[P1.at[0]], rows1, sem1).wait()
        pltpu.sync_copy(rows1, agg.at[P1.at[1]], add=True)
        pltpu.async_copy(scd.at[wid, jnp.minimum(j1 + 2, NBATCH - 1)],
                         P1, semI1)
        pltpu.make_async_copy(scd.at[wid, 0], P0, semI0).wait()
        pltpu.async_copy(h2.at[P0.at[0]], rows0, sem0)
        pltpu.make_async_copy(scd.at[wid, 0], P1, semI1).wait()
        pltpu.async_copy(h2.at[P1.at[0]], rows1, sem1)
        return _
    lax.fori_loop(0, NBATCH // 2, step, None)
    pltpu.make_async_copy(h2.at[P0.at[0]], rows0, sem0).wait()
    pltpu.make_async_copy(h2.at[P1.at[0]], rows1, sem1).wait()

    plsc.subcore_barrier()
    pltpu.sync_copy(agg.at[pl.ds(row0, ROWS_PT)],
                    outp.at[c, pl.ds(row0, ROWS_PT)])


_msg_call = pl.kernel(
    _msg_body,
    out_type=jax.ShapeDtypeStruct((NC, N_PAD, D), jnp.float32),
    mesh=_mesh,
    compiler_params=pltpu.CompilerParams(use_tc_tiling_on_sc=False, needs_layout_passes=False),
    scratch_types=[
        pltpu.VMEM((2, BATCH), jnp.int32),
        pltpu.VMEM((2, BATCH), jnp.int32),
        pltpu.VMEM((BATCH, D), jnp.float32),
        pltpu.VMEM((BATCH, D), jnp.float32),
        pltpu.VMEM((_DR, D), jnp.float32),
        pltpu.VMEM_SHARED((N_PAD, D), jnp.float32),
        pltpu.SemaphoreType.DMA,
        pltpu.SemaphoreType.DMA,
        pltpu.SemaphoreType.DMA,
        pltpu.SemaphoreType.DMA,
        pltpu.SemaphoreType.DMA,
        pltpu.SemaphoreType.DMA,
    ],
)


def _out_body(agg_ref, nd_ref, b_ref, o_ref):
    a = agg_ref[0] + agg_ref[1]
    o_ref[...] = jnp.maximum(a * nd_ref[...] + b_ref[...], 0.0)


_OUT_ROWS = 1000

_out_call = pl.pallas_call(
    _out_body,
    grid=(N // _OUT_ROWS,),
    in_specs=[
        pl.BlockSpec((NC, _OUT_ROWS, D), lambda i: (0, i, 0)),
        pl.BlockSpec((_OUT_ROWS, 1), lambda i: (i, 0)),
        pl.BlockSpec((1, D), lambda i: (0, 0)),
    ],
    out_specs=pl.BlockSpec((_OUT_ROWS, D), lambda i: (i, 0)),
    out_shape=jax.ShapeDtypeStruct((N, D), jnp.float32),
)


@jax.jit
def kernel(x, edge_index, W, b):
    assert x.shape == (N, D) and edge_index.shape == (2, E)
    e32 = edge_index.astype(jnp.int32)
    eidx_a = e32.reshape(2, NW, EPT)
    scd = jnp.stack(
        [e32[0].reshape(NW, NBATCH, BATCH), e32[1].reshape(NW, NBATCH, BATCH)],
        axis=2)

    deg = _deg_call(eidx_a)
    h2, nd = _feat_call(deg, x, W)
    aggp = _msg_call(scd, h2)
    out = _out_call(aggp, nd, b.reshape(1, D))
    return out


# speedup vs baseline: 1.2534x; 1.0082x over previous
import functools

import jax
import jax.numpy as jnp
from jax import lax
from jax.experimental import pallas as pl
from jax.experimental.pallas import tpu as pltpu
from jax.experimental.pallas import tpu_sc as plsc

N = 10000
E = 320000
D = 128
N_PAD = 10240
NC = 2
NS = 16
NW = NC * NS
EPT = E // NW
BATCH = 125
NBATCH = EPT // BATCH
HHALF = N_PAD // 2
ROWS_PT = N_PAD // NS
RED_W = 2 * N_PAD // NS

_mesh = plsc.VectorSubcoreMesh(
    core_axis_name="c", subcore_axis_name="s", num_cores=NC, num_subcores=NS)


def _zeros16():
    return jnp.zeros((16,), jnp.float32)


_RCH = 320


def _deg_body(eidx, deg_out, idx_v, hist, seg_v, tmp_v, acc_v, shdeg):
    c = lax.axis_index("c")
    s = lax.axis_index("s")
    wid = s * NC + c
    lane = lax.iota(jnp.int32, 16)
    lane8 = lane & 7
    m_lo = lane < 8
    m_hi = lane >= 8
    ones = jnp.ones((16,), jnp.float32)

    for arr in range(2):
        pltpu.sync_copy(eidx.at[arr, wid], idx_v)
        def zh(j, _):
            for l in range(8):
                hist[l, pl.ds(j * 16, 16)] = _zeros16()
            return _
        lax.fori_loop(0, N_PAD // 16, zh, None)

        def sc(i, _):
            for u in range(5):
                idx = idx_v[pl.ds((i * 5 + u) * 16, 16)]
                plsc.addupdate_scatter(hist, [lane8, idx], ones, mask=m_lo)
                plsc.addupdate_scatter(hist, [lane8, idx], ones, mask=m_hi)
            return _
        lax.fori_loop(0, EPT // 80, sc, None)

        def rd(j, _):
            acc = hist[0, pl.ds(j * 16, 16)]
            for l in range(1, 8):
                acc = acc + hist[l, pl.ds(j * 16, 16)]
            seg_v[pl.ds(j * 16, 16)] = acc
            return _
        lax.fori_loop(0, N_PAD // 16, rd, None)
        pltpu.sync_copy(seg_v, shdeg.at[s, pl.ds(arr * N_PAD, N_PAD)])

    plsc.subcore_barrier()

    col0 = s * RED_W
    for ch in range(RED_W // _RCH):
        pltpu.sync_copy(shdeg.at[:, pl.ds(col0 + ch * _RCH, _RCH)], tmp_v)

        def ad(j, _):
            acc = tmp_v[0, pl.ds(j * 16, 16)]
            for t in range(1, NS):
                acc = acc + tmp_v[t, pl.ds(j * 16, 16)]
            acc_v[pl.ds(ch * _RCH + j * 16, 16)] = acc
            return _
        lax.fori_loop(0, _RCH // 16, ad, None)

    pltpu.sync_copy(acc_v, deg_out.at[c, pl.ds(col0, RED_W)])


_deg_call = pl.kernel(
    _deg_body,
    out_type=jax.ShapeDtypeStruct((NC, 2 * N_PAD), jnp.float32),
    mesh=_mesh,
    compiler_params=pltpu.CompilerParams(use_tc_tiling_on_sc=False, needs_layout_passes=False),
    scratch_types=[
        pltpu.VMEM((EPT,), jnp.int32),
        pltpu.VMEM((8, N_PAD), jnp.float32),
        pltpu.VMEM((N_PAD,), jnp.float32),
        pltpu.VMEM((NS, _RCH), jnp.float32),
        pltpu.VMEM((RED_W,), jnp.float32),
        pltpu.VMEM_SHARED((NS, 2 * N_PAD), jnp.float32),
    ],
)


def _feat_body(deg_ref, x_ref, w_ref, h2_ref, nd_ref):
    dg = deg_ref[0, :] + deg_ref[1, :]
    dsrc = dg[:N_PAD]
    ddst = dg[N_PAD:]
    ns = jnp.where(dsrc > 0, lax.rsqrt(jnp.maximum(dsrc, 1.0)), 0.0)
    nd = jnp.where(ddst > 0, lax.rsqrt(jnp.maximum(ddst, 1.0)), 0.0)
    h = x_ref[...] * ns[:N, None]
    h2_ref[:N, :] = jnp.dot(h, w_ref[...], preferred_element_type=jnp.float32)
    h2_ref[N:, :] = jnp.zeros((N_PAD - N, D), jnp.float32)
    nd_ref[...] = nd[:, None]


_feat_call = pl.pallas_call(
    _feat_body,
    out_shape=(
        jax.ShapeDtypeStruct((N_PAD, D), jnp.float32),
        jax.ShapeDtypeStruct((N_PAD, 1), jnp.float32),
    ),
)


_DR = 32


def _msg_body(scd, h2, outp, P0, P1, rows0, rows1, dbuf,
              agg, sem0, sem1, semI0, semI1, semS0, semS1):
    c = lax.axis_index("c")
    s = lax.axis_index("s")
    wid = s * NC + c

    def zb(r, _):
        for k in range(D // 16):
            dbuf[r, pl.ds(k * 16, 16)] = _zeros16()
        return _
    lax.fori_loop(0, _DR, zb, None)
    row0 = s * ROWS_PT
    for k in range(ROWS_PT // _DR):
        pltpu.async_copy(dbuf, agg.at[pl.ds(row0 + k * _DR, _DR)], semI0)
    for k in range(ROWS_PT // _DR):
        pltpu.make_async_copy(dbuf, agg.at[pl.ds(row0 + k * _DR, _DR)],
                              semI0).wait()
    plsc.subcore_barrier()

    pltpu.sync_copy(scd.at[wid, 0], P0)
    pltpu.sync_copy(scd.at[wid, 1], P1)
    pltpu.async_copy(h2.at[P0.at[0]], rows0, sem0)
    pltpu.async_copy(h2.at[P1.at[0]], rows1, sem1)

    def step(g, _):
        j0 = g * 2
        j1 = j0 + 1
        pltpu.make_async_copy(h2.at[P0.at[0]], rows0, sem0).wait()
        pltpu.sync_copy(rows0, agg.at[P0.at[1]], add=True)
        pltpu.async_copy(scd.at[wid, jnp.minimum(j0 + 2, NBATCH - 1)],
                         P0, semI0)
        pltpu.make_async_copy(h2.at[P1.at[0]], rows1, sem1).wait()
        pltpu.sync_copy(rows1, agg.at[P1.at[1]], add=True)
        pltpu.async_copy(scd.at[wid, jnp.minimum(j1 + 2, NBATCH - 1)],
                         P1, semI1)
        pltpu.make_async_copy(scd.at[wid, 0], P0, semI0).wait()
        pltpu.async_copy(h2.at[P0.at[0]], rows0, sem0)
        pltpu.make_async_copy(scd.at[wid, 0], P1, semI1).wait()
        pltpu.async_copy(h2.at[P1.at[0]], rows1, sem1)
        return _
    lax.fori_loop(0, NBATCH // 2, step, None)
    pltpu.make_async_copy(h2.at[P0.at[0]], rows0, sem0).wait()
    pltpu.make_async_copy(h2.at[P1.at[0]], rows1, sem1).wait()

    plsc.subcore_barrier()
    pltpu.sync_copy(agg.at[pl.ds(row0, ROWS_PT)],
                    outp.at[c, pl.ds(row0, ROWS_PT)])


_msg_call = pl.kernel(
    _msg_body,
    out_type=jax.ShapeDtypeStruct((NC, N_PAD, D), jnp.float32),
    mesh=_mesh,
    compiler_params=pltpu.CompilerParams(use_tc_tiling_on_sc=False, needs_layout_passes=False),
    scratch_types=[
        pltpu.VMEM((2, BATCH), jnp.int32),
        pltpu.VMEM((2, BATCH), jnp.int32),
        pltpu.VMEM((BATCH, D), jnp.float32),
        pltpu.VMEM((BATCH, D), jnp.float32),
        pltpu.VMEM((_DR, D), jnp.float32),
        pltpu.VMEM_SHARED((N_PAD, D), jnp.float32),
        pltpu.SemaphoreType.DMA,
        pltpu.SemaphoreType.DMA,
        pltpu.SemaphoreType.DMA,
        pltpu.SemaphoreType.DMA,
        pltpu.SemaphoreType.DMA,
        pltpu.SemaphoreType.DMA,
    ],
)


def _out_body(agg_ref, nd_ref, b_ref, o_ref):
    a = agg_ref[0] + agg_ref[1]
    o_ref[...] = jnp.maximum(a * nd_ref[...] + b_ref[...], 0.0)


_OUT_ROWS = 1000

_out_call = pl.pallas_call(
    _out_body,
    grid=(N // _OUT_ROWS,),
    in_specs=[
        pl.BlockSpec((NC, _OUT_ROWS, D), lambda i: (0, i, 0)),
        pl.BlockSpec((_OUT_ROWS, 1), lambda i: (i, 0)),
        pl.BlockSpec((1, D), lambda i: (0, 0)),
    ],
    out_specs=pl.BlockSpec((_OUT_ROWS, D), lambda i: (i, 0)),
    out_shape=jax.ShapeDtypeStruct((N, D), jnp.float32),
)


@jax.jit
def kernel(x, edge_index, W, b):
    assert x.shape == (N, D) and edge_index.shape == (2, E)
    e32 = edge_index.astype(jnp.int32)
    eidx_a = e32.reshape(2, NW, EPT)
    scd = jnp.stack(
        [e32[0].reshape(NW, NBATCH, BATCH), e32[1].reshape(NW, NBATCH, BATCH)],
        axis=2)

    deg = _deg_call(eidx_a)
    h2, nd = _feat_call(deg, x, W)
    aggp = _msg_call(scd, h2)
    out = _out_call(aggp, nd, b.reshape(1, D))
    return out
